# Initial kernel scaffold; baseline (speedup 1.0000x reference)
import jax
import jax.numpy as jnp
from jax.experimental import pallas as pl


def kernel(coords, vals, mask, W1, b1, W2, b2, W3, b3):
    raise NotImplementedError("write your pallas kernel here")


# speedup vs baseline: 6.2292x; 6.2292x over previous
import jax
import jax.numpy as jnp
from jax import lax
from jax.experimental import pallas as pl
from jax.experimental.pallas import tpu as pltpu

_BS, _N, _C = 8, 4096, 128
_HID, _OUT = 64, 16
_U = 16


def _body(orb_ref, maskf_ref, vals_ref, W1_ref, b1_ref, W2_ref, b2_ref,
          W3_ref, b3_ref, out_ref):
    orb = orb_ref[0]
    maskf = maskf_ref[0]
    orb_b = jnp.broadcast_to(orb, (_U, _N))
    row_u = lax.broadcasted_iota(jnp.int32, (_U, _N), 0)
    ohT = jnp.where(orb_b == row_u, jnp.broadcast_to(maskf, (_U, _N)), 0.0)
    agg = lax.dot_general(ohT, vals_ref[0],
                          (((1,), (0,)), ((), ())),
                          preferred_element_type=jnp.float32)
    rowsum = jnp.sum(agg, axis=1, keepdims=True)
    empty = rowsum == 0.0
    h = jax.nn.relu(jnp.dot(agg, W1_ref[...],
                            preferred_element_type=jnp.float32) + b1_ref[...])
    h = jax.nn.relu(jnp.dot(h, W2_ref[...],
                            preferred_element_type=jnp.float32) + b2_ref[...])
    t = jnp.dot(h, W3_ref[...], preferred_element_type=jnp.float32) + b3_ref[...]
    t = jnp.where(empty, 0.0, t)
    out_ref[0] = jnp.sum(t, axis=0, keepdims=True)


def kernel(coords, vals, mask, W1, b1, W2, b2, W3, b3):
    orb_ids = coords[:, :, 1, 1].astype(jnp.int32).reshape(_BS, 1, _N)
    maskf = mask.astype(jnp.float32).reshape(_BS, 1, _N)
    b1r = b1.reshape(1, _HID)
    b2r = b2.reshape(1, _HID)
    b3r = b3.reshape(1, _OUT)

    out = pl.pallas_call(
        _body,
        grid=(_BS,),
        in_specs=[
            pl.BlockSpec((1, 1, _N), lambda b: (b, 0, 0)),
            pl.BlockSpec((1, 1, _N), lambda b: (b, 0, 0)),
            pl.BlockSpec((1, _N, _C), lambda b: (b, 0, 0)),
            pl.BlockSpec((_C, _HID), lambda b: (0, 0)),
            pl.BlockSpec((1, _HID), lambda b: (0, 0)),
            pl.BlockSpec((_HID, _HID), lambda b: (0, 0)),
            pl.BlockSpec((1, _HID), lambda b: (0, 0)),
            pl.BlockSpec((_HID, _OUT), lambda b: (0, 0)),
            pl.BlockSpec((1, _OUT), lambda b: (0, 0)),
        ],
        out_specs=pl.BlockSpec((1, 1, _OUT), lambda b: (b, 0, 0)),
        out_shape=jax.ShapeDtypeStruct((_BS, 1, _OUT), jnp.float32),
    )(orb_ids, maskf, vals, W1, b1r, W2, b2r, W3, b3r)
    return out.reshape(_BS, _OUT)
